# Initial kernel scaffold; baseline (speedup 1.0000x reference)
import jax
import jax.numpy as jnp
from jax.experimental import pallas as pl


def kernel(x, edge_index, W1, b1, W2, b2):
    raise NotImplementedError("write your pallas kernel here")


# speedup vs baseline: 25.5439x; 25.5439x over previous
import functools

import jax
import jax.numpy as jnp
from jax import lax
from jax.experimental import pallas as pl
from jax.experimental.pallas import tpu as pltpu
from jax.experimental.pallas import tpu_sc as plsc

_N = 10000
_E = 320000
_NC = 2
_NS = 16
_NW = _NC * _NS
_EPT = _E // _NW
_CH = 80
_NCHUNK = _EPT // _CH
_RZ = 1000
_NZT = _N // _RZ

_mesh = plsc.VectorSubcoreMesh(core_axis_name="c", subcore_axis_name="s")
_sc_params = pltpu.CompilerParams(use_tc_tiling_on_sc=False)


def _deg_body(dst_hbm, ones_hbm, zeros_hbm, out_hbm, didx, ones_v, acc):
  c = lax.axis_index("c")
  s = lax.axis_index("s")
  wid = c * _NS + s
  pltpu.sync_copy(dst_hbm.at[wid], didx)
  pltpu.sync_copy(ones_hbm, ones_v)

  @pl.when(s < _NZT)
  def _():
    pltpu.sync_copy(zeros_hbm, acc.at[pl.ds(s * _RZ, _RZ)])

  plsc.subcore_barrier()

  def body(j, carry):
    pltpu.sync_copy(ones_v, acc.at[didx.at[j]], add=True)
    return carry

  lax.fori_loop(0, _NCHUNK, body, 0)
  plsc.subcore_barrier()

  @pl.when(s < _NZT)
  def _():
    pltpu.sync_copy(acc.at[pl.ds(s * _RZ, _RZ)],
                    out_hbm.at[c, pl.ds(s * _RZ, _RZ)])


_sc_deg = functools.partial(
    pl.kernel,
    out_type=jax.ShapeDtypeStruct((_NC, _N, 1), jnp.float32),
    mesh=_mesh,
    compiler_params=_sc_params,
    scratch_types=[
        pltpu.VMEM((_NCHUNK, _CH), jnp.int32),
        pltpu.VMEM((_CH, 1), jnp.float32),
        pltpu.MemorySpace.VMEM_SHARED((_N, 1), jnp.float32),
    ],
)(_deg_body)


def _make_edge(d):
  def body(h_hbm, src_hbm, dst_hbm, zeros_hbm, out_hbm,
           sidx, didx, rows, acc, sem):
    c = lax.axis_index("c")
    s = lax.axis_index("s")
    wid = c * _NS + s
    pltpu.sync_copy(src_hbm.at[wid], sidx)
    pltpu.sync_copy(dst_hbm.at[wid], didx)

    @pl.when(s < _NZT)
    def _():
      pltpu.sync_copy(zeros_hbm, acc.at[pl.ds(s * _RZ, _RZ)])

    plsc.subcore_barrier()

    def step(j, carry):
      pltpu.async_copy(h_hbm.at[sidx.at[j]], rows, sem).wait()
      pltpu.sync_copy(rows, acc.at[didx.at[j]], add=True)
      return carry

    lax.fori_loop(0, _NCHUNK, step, 0)
    plsc.subcore_barrier()

    @pl.when(s < _NZT)
    def _():
      pltpu.sync_copy(acc.at[pl.ds(s * _RZ, _RZ)],
                      out_hbm.at[c, pl.ds(s * _RZ, _RZ)])

  return functools.partial(
      pl.kernel,
      out_type=jax.ShapeDtypeStruct((_NC, _N, d), jnp.float32),
      mesh=_mesh,
      compiler_params=_sc_params,
      scratch_types=[
          pltpu.VMEM((_NCHUNK, _CH), jnp.int32),
          pltpu.VMEM((_NCHUNK, _CH), jnp.int32),
          pltpu.VMEM((_CH, d), jnp.float32),
          pltpu.MemorySpace.VMEM_SHARED((_N, d), jnp.float32),
          pltpu.SemaphoreType.DMA,
      ],
  )(body)


_sc_edge64 = _make_edge(64)
_sc_edge32 = _make_edge(32)


def _tc1_body(degp_ref, x_ref, w1_ref, dis_ref, h1s_ref):
  deg = degp_ref[0] + degp_ref[1] + 1.0
  dis = lax.rsqrt(deg)
  dis_ref[...] = dis
  h = jnp.dot(x_ref[...], w1_ref[...], preferred_element_type=jnp.float32)
  h1s_ref[...] = h * dis


_tc1 = pl.pallas_call(
    _tc1_body,
    out_shape=[
        jax.ShapeDtypeStruct((_N, 1), jnp.float32),
        jax.ShapeDtypeStruct((_N, 64), jnp.float32),
    ],
)


def _tc2_body(acc_ref, h1s_ref, dis_ref, b1_ref, w2_ref, h2s_ref):
  dis = dis_ref[...]
  z = dis * (acc_ref[0] + acc_ref[1] + h1s_ref[...]) + b1_ref[...]
  z = jnp.maximum(z, 0.0)
  h2 = jnp.dot(z, w2_ref[...], preferred_element_type=jnp.float32)
  h2s_ref[...] = h2 * dis


_tc2 = pl.pallas_call(
    _tc2_body,
    out_shape=jax.ShapeDtypeStruct((_N, 32), jnp.float32),
)


def _tc3_body(acc_ref, h2s_ref, dis_ref, b2_ref, out_ref):
  out_ref[...] = (dis_ref[...] * (acc_ref[0] + acc_ref[1] + h2s_ref[...])
                  + b2_ref[...])


_tc3 = pl.pallas_call(
    _tc3_body,
    out_shape=jax.ShapeDtypeStruct((_N, 32), jnp.float32),
)


@jax.jit
def kernel(x, edge_index, W1, b1, W2, b2):
  src = edge_index[0].reshape(_NW, _NCHUNK, _CH)
  dst = edge_index[1].reshape(_NW, _NCHUNK, _CH)
  ones = jnp.ones((_CH, 1), jnp.float32)
  z1 = jnp.zeros((_RZ, 1), jnp.float32)
  z64 = jnp.zeros((_RZ, 64), jnp.float32)
  z32 = jnp.zeros((_RZ, 32), jnp.float32)

  degp = _sc_deg(dst, ones, z1)
  dis, h1s = _tc1(degp, x, W1)
  acc1 = _sc_edge64(h1s, src, dst, z64)
  h2s = _tc2(acc1, h1s, dis, b1.reshape(1, 64), W2)
  acc2 = _sc_edge32(h2s, src, dst, z32)
  return _tc3(acc2, h2s, dis, b2.reshape(1, 32))


# speedup vs baseline: 32.7808x; 1.2833x over previous
import functools

import jax
import jax.numpy as jnp
from jax import lax
from jax.experimental import pallas as pl
from jax.experimental.pallas import tpu as pltpu
from jax.experimental.pallas import tpu_sc as plsc

_N = 10000
_E = 320000
_NC = 2
_NS = 16
_NW = _NC * _NS
_EPT = _E // _NW
_CH = 80
_NCHUNK = _EPT // _CH
_RZ = 1000
_NZT = _N // _RZ

_mesh = plsc.VectorSubcoreMesh(core_axis_name="c", subcore_axis_name="s")
_sc_params = pltpu.CompilerParams(use_tc_tiling_on_sc=False)


def _deg_body(dst_hbm, ones_hbm, zeros_hbm, out_hbm, didx, ones_v, acc):
  c = lax.axis_index("c")
  s = lax.axis_index("s")
  wid = c * _NS + s
  pltpu.sync_copy(dst_hbm.at[wid], didx)
  pltpu.sync_copy(ones_hbm, ones_v)

  @pl.when(s < _NZT)
  def _():
    pltpu.sync_copy(zeros_hbm, acc.at[pl.ds(s * _RZ, _RZ)])

  plsc.subcore_barrier()

  def body(j, carry):
    pltpu.sync_copy(ones_v, acc.at[didx.at[j]], add=True)
    return carry

  lax.fori_loop(0, _NCHUNK, body, 0)
  plsc.subcore_barrier()

  @pl.when(s < _NZT)
  def _():
    pltpu.sync_copy(acc.at[pl.ds(s * _RZ, _RZ)],
                    out_hbm.at[c, pl.ds(s * _RZ, _RZ)])


_sc_deg = functools.partial(
    pl.kernel,
    out_type=jax.ShapeDtypeStruct((_NC, _N, 1), jnp.float32),
    mesh=_mesh,
    compiler_params=_sc_params,
    scratch_types=[
        pltpu.VMEM((_NCHUNK, _CH), jnp.int32),
        pltpu.VMEM((_CH, 1), jnp.float32),
        pltpu.MemorySpace.VMEM_SHARED((_N, 1), jnp.float32),
    ],
)(_deg_body)


def _make_edge(d):
  def body(h_hbm, src_hbm, dst_hbm, zeros_hbm, out_hbm,
           sidx, didx, rows_a, rows_b, acc, sem_a, sem_b):
    c = lax.axis_index("c")
    s = lax.axis_index("s")
    wid = c * _NS + s
    pltpu.sync_copy(src_hbm.at[wid], sidx)
    pltpu.sync_copy(dst_hbm.at[wid], didx)

    @pl.when(s < _NZT)
    def _():
      pltpu.sync_copy(zeros_hbm, acc.at[pl.ds(s * _RZ, _RZ)])

    plsc.subcore_barrier()

    def step(g, carry):
      j0 = 2 * g
      j1 = 2 * g + 1
      da = pltpu.async_copy(h_hbm.at[sidx.at[j0]], rows_a, sem_a)
      db = pltpu.async_copy(h_hbm.at[sidx.at[j1]], rows_b, sem_b)
      da.wait()
      pltpu.sync_copy(rows_a, acc.at[didx.at[j0]], add=True)
      db.wait()
      pltpu.sync_copy(rows_b, acc.at[didx.at[j1]], add=True)
      return carry

    lax.fori_loop(0, _NCHUNK // 2, step, 0)
    jt = _NCHUNK - 1
    pltpu.async_copy(h_hbm.at[sidx.at[jt]], rows_a, sem_a).wait()
    pltpu.sync_copy(rows_a, acc.at[didx.at[jt]], add=True)
    plsc.subcore_barrier()

    @pl.when(s < _NZT)
    def _():
      pltpu.sync_copy(acc.at[pl.ds(s * _RZ, _RZ)],
                      out_hbm.at[c, pl.ds(s * _RZ, _RZ)])

  return functools.partial(
      pl.kernel,
      out_type=jax.ShapeDtypeStruct((_NC, _N, d), jnp.float32),
      mesh=_mesh,
      compiler_params=_sc_params,
      scratch_types=[
          pltpu.VMEM((_NCHUNK, _CH), jnp.int32),
          pltpu.VMEM((_NCHUNK, _CH), jnp.int32),
          pltpu.VMEM((_CH, d), jnp.float32),
          pltpu.VMEM((_CH, d), jnp.float32),
          pltpu.MemorySpace.VMEM_SHARED((_N, d), jnp.float32),
          pltpu.SemaphoreType.DMA,
          pltpu.SemaphoreType.DMA,
      ],
  )(body)


_sc_edge64 = _make_edge(64)
_sc_edge32 = _make_edge(32)


def _tc1_body(degp_ref, x_ref, w1_ref, dis_ref, h1s_ref):
  deg = degp_ref[0] + degp_ref[1] + 1.0
  dis = lax.rsqrt(deg)
  dis_ref[...] = dis
  h = jnp.dot(x_ref[...], w1_ref[...], preferred_element_type=jnp.float32)
  h1s_ref[...] = h * dis


_tc1 = pl.pallas_call(
    _tc1_body,
    out_shape=[
        jax.ShapeDtypeStruct((_N, 1), jnp.float32),
        jax.ShapeDtypeStruct((_N, 64), jnp.float32),
    ],
)


def _tc2_body(acc_ref, h1s_ref, dis_ref, b1_ref, w2_ref, h2s_ref):
  dis = dis_ref[...]
  z = dis * (acc_ref[0] + acc_ref[1] + h1s_ref[...]) + b1_ref[...]
  z = jnp.maximum(z, 0.0)
  h2 = jnp.dot(z, w2_ref[...], preferred_element_type=jnp.float32)
  h2s_ref[...] = h2 * dis


_tc2 = pl.pallas_call(
    _tc2_body,
    out_shape=jax.ShapeDtypeStruct((_N, 32), jnp.float32),
)


def _tc3_body(acc_ref, h2s_ref, dis_ref, b2_ref, out_ref):
  out_ref[...] = (dis_ref[...] * (acc_ref[0] + acc_ref[1] + h2s_ref[...])
                  + b2_ref[...])


_tc3 = pl.pallas_call(
    _tc3_body,
    out_shape=jax.ShapeDtypeStruct((_N, 32), jnp.float32),
)


@jax.jit
def kernel(x, edge_index, W1, b1, W2, b2):
  src = edge_index[0].reshape(_NW, _NCHUNK, _CH)
  dst = edge_index[1].reshape(_NW, _NCHUNK, _CH)
  ones = jnp.ones((_CH, 1), jnp.float32)
  z1 = jnp.zeros((_RZ, 1), jnp.float32)
  z64 = jnp.zeros((_RZ, 64), jnp.float32)
  z32 = jnp.zeros((_RZ, 32), jnp.float32)

  degp = _sc_deg(dst, ones, z1)
  dis, h1s = _tc1(degp, x, W1)
  acc1 = _sc_edge64(h1s, src, dst, z64)
  h2s = _tc2(acc1, h1s, dis, b1.reshape(1, 64), W2)
  acc2 = _sc_edge32(h2s, src, dst, z32)
  return _tc3(acc2, h2s, dis, b2.reshape(1, 32))


# speedup vs baseline: 33.7242x; 1.0288x over previous
import functools

import jax
import jax.numpy as jnp
from jax import lax
from jax.experimental import pallas as pl
from jax.experimental.pallas import tpu as pltpu
from jax.experimental.pallas import tpu_sc as plsc

_N = 10000
_E = 320000
_NC = 2
_NS = 16
_NW = _NC * _NS
_EPT = _E // _NW
_CH = 80
_NCHUNK = _EPT // _CH
_RZ = 1000
_NZT = _N // _RZ

_mesh = plsc.VectorSubcoreMesh(core_axis_name="c", subcore_axis_name="s")
_sc_params = pltpu.CompilerParams(use_tc_tiling_on_sc=False)


def _deg_body(dst_hbm, ones_hbm, zeros_hbm, out_hbm, didx, ones_v, acc):
  c = lax.axis_index("c")
  s = lax.axis_index("s")
  wid = c * _NS + s
  pltpu.sync_copy(dst_hbm.at[wid], didx)
  pltpu.sync_copy(ones_hbm, ones_v)

  @pl.when(s < _NZT)
  def _():
    pltpu.sync_copy(zeros_hbm, acc.at[pl.ds(s * _RZ, _RZ)])

  plsc.subcore_barrier()

  def body(j, carry):
    pltpu.sync_copy(ones_v, acc.at[didx.at[j]], add=True)
    return carry

  lax.fori_loop(0, _NCHUNK, body, 0)
  plsc.subcore_barrier()

  @pl.when(s < _NZT)
  def _():
    pltpu.sync_copy(acc.at[pl.ds(s * _RZ, _RZ)],
                    out_hbm.at[c, pl.ds(s * _RZ, _RZ)])


_sc_deg = functools.partial(
    pl.kernel,
    out_type=jax.ShapeDtypeStruct((_NC, _N, 1), jnp.float32),
    mesh=_mesh,
    compiler_params=_sc_params,
    scratch_types=[
        pltpu.VMEM((_NCHUNK, _CH), jnp.int32),
        pltpu.VMEM((_CH, 1), jnp.float32),
        pltpu.MemorySpace.VMEM_SHARED((_N, 1), jnp.float32),
    ],
)(_deg_body)


def _make_edge(d):
  def body(h_hbm, src_hbm, dst_hbm, zeros_hbm, out_hbm,
           sidx, didx, rows_a, rows_b, acc, sem_a, sem_b, sem_c, sem_d):
    c = lax.axis_index("c")
    s = lax.axis_index("s")
    wid = c * _NS + s
    pltpu.sync_copy(src_hbm.at[wid], sidx)
    pltpu.sync_copy(dst_hbm.at[wid], didx)

    @pl.when(s < _NZT)
    def _():
      pltpu.sync_copy(zeros_hbm, acc.at[pl.ds(s * _RZ, _RZ)])

    plsc.subcore_barrier()

    def step(g, carry):
      j0 = 2 * g
      j1 = 2 * g + 1
      da = pltpu.async_copy(h_hbm.at[sidx.at[j0]], rows_a, sem_a)
      db = pltpu.async_copy(h_hbm.at[sidx.at[j1]], rows_b, sem_b)
      da.wait()
      sa = pltpu.async_copy(rows_a, acc.at[didx.at[j0]], sem_c, add=True)
      db.wait()
      sb = pltpu.async_copy(rows_b, acc.at[didx.at[j1]], sem_d, add=True)
      sa.wait()
      sb.wait()
      return carry

    lax.fori_loop(0, _NCHUNK // 2, step, 0)
    jt = _NCHUNK - 1
    pltpu.async_copy(h_hbm.at[sidx.at[jt]], rows_a, sem_a).wait()
    pltpu.sync_copy(rows_a, acc.at[didx.at[jt]], add=True)
    plsc.subcore_barrier()

    @pl.when(s < _NZT)
    def _():
      pltpu.sync_copy(acc.at[pl.ds(s * _RZ, _RZ)],
                      out_hbm.at[c, pl.ds(s * _RZ, _RZ)])

  return functools.partial(
      pl.kernel,
      out_type=jax.ShapeDtypeStruct((_NC, _N, d), jnp.float32),
      mesh=_mesh,
      compiler_params=_sc_params,
      scratch_types=[
          pltpu.VMEM((_NCHUNK, _CH), jnp.int32),
          pltpu.VMEM((_NCHUNK, _CH), jnp.int32),
          pltpu.VMEM((_CH, d), jnp.float32),
          pltpu.VMEM((_CH, d), jnp.float32),
          pltpu.MemorySpace.VMEM_SHARED((_N, d), jnp.float32),
          pltpu.SemaphoreType.DMA,
          pltpu.SemaphoreType.DMA,
          pltpu.SemaphoreType.DMA,
          pltpu.SemaphoreType.DMA,
      ],
  )(body)


_sc_edge64 = _make_edge(64)
_sc_edge32 = _make_edge(32)


def _tc1_body(degp_ref, x_ref, w1_ref, dis_ref, h1s_ref):
  deg = degp_ref[0] + degp_ref[1] + 1.0
  dis = lax.rsqrt(deg)
  dis_ref[...] = dis
  h = jnp.dot(x_ref[...], w1_ref[...], preferred_element_type=jnp.float32)
  h1s_ref[...] = h * dis


_tc1 = pl.pallas_call(
    _tc1_body,
    out_shape=[
        jax.ShapeDtypeStruct((_N, 1), jnp.float32),
        jax.ShapeDtypeStruct((_N, 64), jnp.float32),
    ],
)


def _tc2_body(acc_ref, h1s_ref, dis_ref, b1_ref, w2_ref, h2s_ref):
  dis = dis_ref[...]
  z = dis * (acc_ref[0] + acc_ref[1] + h1s_ref[...]) + b1_ref[...]
  z = jnp.maximum(z, 0.0)
  h2 = jnp.dot(z, w2_ref[...], preferred_element_type=jnp.float32)
  h2s_ref[...] = h2 * dis


_tc2 = pl.pallas_call(
    _tc2_body,
    out_shape=jax.ShapeDtypeStruct((_N, 32), jnp.float32),
)


def _tc3_body(acc_ref, h2s_ref, dis_ref, b2_ref, out_ref):
  out_ref[...] = (dis_ref[...] * (acc_ref[0] + acc_ref[1] + h2s_ref[...])
                  + b2_ref[...])


_tc3 = pl.pallas_call(
    _tc3_body,
    out_shape=jax.ShapeDtypeStruct((_N, 32), jnp.float32),
)


@jax.jit
def kernel(x, edge_index, W1, b1, W2, b2):
  src = edge_index[0].reshape(_NW, _NCHUNK, _CH)
  dst = edge_index[1].reshape(_NW, _NCHUNK, _CH)
  ones = jnp.ones((_CH, 1), jnp.float32)
  z1 = jnp.zeros((_RZ, 1), jnp.float32)
  z64 = jnp.zeros((_RZ, 64), jnp.float32)
  z32 = jnp.zeros((_RZ, 32), jnp.float32)

  degp = _sc_deg(dst, ones, z1)
  dis, h1s = _tc1(degp, x, W1)
  acc1 = _sc_edge64(h1s, src, dst, z64)
  h2s = _tc2(acc1, h1s, dis, b1.reshape(1, 64), W2)
  acc2 = _sc_edge32(h2s, src, dst, z32)
  return _tc3(acc2, h2s, dis, b2.reshape(1, 32))
